# baseline (speedup 1.0000x reference)
import functools

import jax
import jax.numpy as jnp
from jax import lax
from jax.experimental import pallas as pl
from jax.experimental.pallas import tpu as pltpu
from jax.experimental.pallas import tpu_sc as plsc

EMBED = 32
GROUP = 128
SETB = 4
BLKW = EMBED * GROUP
SETW = SETB * BLKW


def _sc_mesh_info():
    info = plsc.get_sparse_core_info()
    return info.num_cores, info.num_subcores


def _detile_table(table):
    NV, NE = table.shape
    n_lines = NV * NE // 128
    TAILV = 576
    nblk = (NV - TAILV) // GROUP
    NC, NS = _sc_mesh_info()
    NW = NC * NS
    per_w = nblk // NW
    n_pairs = per_w // 2
    tail_lines = TAILV * NE // 128

    tabT = jnp.transpose(table)
    tail2 = table[NV - TAILV :].reshape(tail_lines * 128)

    mesh = plsc.VectorSubcoreMesh(core_axis_name="c", subcore_axis_name="s")

    @functools.partial(
        pl.kernel,
        mesh=mesh,
        compiler_params=pltpu.CompilerParams(
            use_tc_tiling_on_sc=True, needs_layout_passes=False
        ),
        out_type=jax.ShapeDtypeStruct((n_lines * 128,), jnp.float32),
        scratch_types=[
            pltpu.VMEM((EMBED, GROUP), jnp.float32),
            pltpu.VMEM((EMBED, GROUP), jnp.float32),
            pltpu.VMEM((BLKW,), jnp.float32),
            pltpu.VMEM((BLKW,), jnp.float32),
            pltpu.VMEM((tail_lines * 128,), jnp.float32),
            pltpu.SemaphoreType.DMA,
            pltpu.SemaphoreType.DMA,
            pltpu.SemaphoreType.DMA,
            pltpu.SemaphoreType.DMA,
        ],
    )
    def det(tabt_hbm, tail_hbm, out_hbm, win0, win1, stg0, stg1, tailbuf,
            gsem0, gsem1, wsem0, wsem1):
        wid = lax.axis_index("s") * NC + lax.axis_index("c")
        base = wid * per_w
        lane = lax.iota(jnp.int32, 16)
        bv = lane * EMBED

        def fire(k, win_p, gsem_p):
            pltpu.async_copy(
                tabt_hbm.at[:, pl.ds((base + k) * GROUP, GROUP)], win_p, gsem_p
            )

        def drain(win_p, gsem_p):
            pltpu.make_async_copy(
                tabt_hbm.at[:, pl.ds(0, GROUP)], win_p, gsem_p
            ).wait()

        def do_blk(it, p, win_p, win_q, stg_p, gsem_p, gsem_q, wsem_p):
            k = it * 2 + p

            if p == 0:
                fire(k + 1, win_q, gsem_q)
            else:
                @pl.when(it < n_pairs - 1)
                def _fire_next():
                    fire(k + 1, win_q, gsem_q)

            drain(win_p, gsem_p)

            @pl.when(it >= 1)
            def _wait_prev_writeback():
                pltpu.make_async_copy(
                    stg_p, out_hbm.at[pl.ds(0, BLKW)], wsem_p
                ).wait()

            def cbody(c, carry):
                coff = c * 16 * EMBED
                for e in range(EMBED):
                    d = win_p[e, pl.ds(c * 16, 16)]
                    plsc.store_scatter(stg_p, [bv + (coff + e)], d)
                return carry

            lax.fori_loop(0, GROUP // 16, cbody, 0)

            pltpu.async_copy(
                stg_p,
                out_hbm.at[pl.ds((base + k) * BLKW, BLKW)],
                wsem_p,
            )

        def body(it, carry):
            do_blk(it, 0, win0, win1, stg0, gsem0, gsem1, wsem0)
            do_blk(it, 1, win1, win0, stg1, gsem1, gsem0, wsem1)
            return carry

        fire(0, win0, gsem0)
        lax.fori_loop(0, n_pairs, body, 0)

        pltpu.make_async_copy(stg0, out_hbm.at[pl.ds(0, BLKW)], wsem0).wait()
        pltpu.make_async_copy(stg1, out_hbm.at[pl.ds(0, BLKW)], wsem1).wait()

        @pl.when(wid == NW - 1)
        def _tail():
            pltpu.sync_copy(tail_hbm, tailbuf)
            pltpu.sync_copy(
                tailbuf,
                out_hbm.at[pl.ds((n_lines - tail_lines) * 128, tail_lines * 128)],
            )

    return det(tabT, tail2)


def kernel(x, table):
    NI, NJ = x.shape
    NV = table.shape[0]
    assert NI % GROUP == 0
    n_blocks = NJ * (NI // GROUP)
    ib_per_j = NI // GROUP

    NC, NS = _sc_mesh_info()
    NW = NC * NS
    assert n_blocks % (NW * SETB) == 0
    g_per_w = n_blocks // NW
    n_sets = g_per_w // SETB
    assert n_sets % 2 == 0
    n_pairs = n_sets // 2

    idx2d = jnp.transpose(x).reshape(n_blocks, GROUP)
    rowtab = _detile_table(table).reshape(NV, EMBED)
    out_words = NJ * EMBED * NI

    mesh = plsc.VectorSubcoreMesh(core_axis_name="c", subcore_axis_name="s")

    @functools.partial(
        pl.kernel,
        mesh=mesh,
        compiler_params=pltpu.CompilerParams(
            use_tc_tiling_on_sc=False, needs_layout_passes=False
        ),
        out_type=jax.ShapeDtypeStruct((out_words,), jnp.float32),
        scratch_types=[
            pltpu.VMEM((g_per_w, GROUP), jnp.int32),
            pltpu.VMEM((SETB, GROUP, EMBED), jnp.float32),
            pltpu.VMEM((SETB, GROUP, EMBED), jnp.float32),
            pltpu.VMEM((SETW,), jnp.float32),
            pltpu.VMEM((SETW,), jnp.float32),
            pltpu.SemaphoreType.DMA,
            pltpu.SemaphoreType.DMA,
            pltpu.SemaphoreType.DMA,
            pltpu.SemaphoreType.DMA,
        ],
    )
    def emb(idx_hbm, table_hbm, out_hbm, idx_v, rows0, rows1, blk0, blk1,
            gsem0, gsem1, wsem0, wsem1):
        wid = lax.axis_index("s") * NC + lax.axis_index("c")
        gbase = wid * g_per_w
        pltpu.sync_copy(idx_hbm.at[pl.ds(gbase, g_per_w)], idx_v)

        lane = lax.iota(jnp.int32, 16)
        b_lo = (lane >> 3) * BLKW + (lane & 7) * GROUP

        def fire_set(s, rows_p, gsem_p):
            for b in range(SETB):
                pltpu.async_copy(
                    table_hbm.at[idx_v.at[s * SETB + b]],
                    rows_p.at[b],
                    gsem_p,
                )

        def drain_set(rows_p, gsem_p):
            for b in range(SETB):
                pltpu.make_async_copy(
                    table_hbm.at[pl.ds(0, GROUP)], rows_p.at[b], gsem_p
                ).wait()

        def do_set(it, p, rows_p, rows_q, blk_p, gsem_p, gsem_q, wsem_p):
            s = it * 2 + p
            g0 = gbase + s * SETB

            if p == 0:
                fire_set(s + 1, rows_q, gsem_q)
            else:
                @pl.when(it < n_pairs - 1)
                def _fire_next():
                    fire_set(s + 1, rows_q, gsem_q)

            drain_set(rows_p, gsem_p)

            @pl.when(it >= 1)
            def _wait_prev_writeback():
                pltpu.make_async_copy(
                    blk_p, out_hbm.at[pl.ds(0, SETW)], wsem_p
                ).wait()

            for b in range(SETB):
                def tbody(i2, carry):
                    off = b * (8 * GROUP) + i2
                    d_lo = rows_p[b, i2, pl.ds(0, 16)]
                    d_hi = rows_p[b, i2, pl.ds(16, 16)]
                    plsc.store_scatter(blk_p, [b_lo + off], d_lo)
                    plsc.store_scatter(blk_p, [b_lo + off + 2 * BLKW], d_hi)
                    return carry

                lax.fori_loop(0, GROUP, tbody, 0)

            j = g0 // ib_per_j
            ib0 = g0 % ib_per_j
            for eb in range(EMBED // 8):
                off = ((j * (EMBED // 8) + eb) * ib_per_j + ib0) * (8 * GROUP)
                pltpu.async_copy(
                    blk_p.at[pl.ds(eb * BLKW, BLKW)],
                    out_hbm.at[pl.ds(off, BLKW)],
                    wsem_p,
                )

        fire_set(0, rows0, gsem0)

        def body(it, carry):
            do_set(it, 0, rows0, rows1, blk0, gsem0, gsem1, wsem0)
            do_set(it, 1, rows1, rows0, blk1, gsem1, gsem0, wsem1)
            return carry

        lax.fori_loop(0, n_pairs, body, 0)

        pltpu.make_async_copy(blk0, out_hbm.at[pl.ds(0, SETW)], wsem0).wait()
        pltpu.make_async_copy(blk1, out_hbm.at[pl.ds(0, SETW)], wsem1).wait()

    out1d = emb(idx2d, rowtab)
    return (
        out1d.reshape(NJ, EMBED // 8, ib_per_j, 8, GROUP)
        .transpose((2, 4, 0, 1, 3))
        .reshape(NI, NJ, EMBED)
    )


# speedup vs baseline: 1.1251x; 1.1251x over previous
import functools

import jax
import jax.numpy as jnp
from jax import lax
from jax.experimental import pallas as pl
from jax.experimental.pallas import tpu as pltpu
from jax.experimental.pallas import tpu_sc as plsc

EMBED = 32
GROUP = 128
SETB = 4
BLKW = EMBED * GROUP
SETW = SETB * BLKW


def kernel(x, table):
    NI, NJ = x.shape
    assert NI % GROUP == 0
    n_blocks = NJ * (NI // GROUP)
    ib_per_j = NI // GROUP

    info = plsc.get_sparse_core_info()
    NC, NS = info.num_cores, info.num_subcores
    NW = NC * NS
    assert n_blocks % (NW * SETB) == 0
    g_per_w = n_blocks // NW
    n_sets = g_per_w // SETB
    assert n_sets % 2 == 0
    n_pairs = n_sets // 2

    idx2d = jnp.transpose(x).reshape(n_blocks, GROUP)
    out_words = NJ * EMBED * NI

    mesh = plsc.VectorSubcoreMesh(core_axis_name="c", subcore_axis_name="s")

    @functools.partial(
        pl.kernel,
        mesh=mesh,
        compiler_params=pltpu.CompilerParams(
            use_tc_tiling_on_sc=False, needs_layout_passes=False
        ),
        out_type=jax.ShapeDtypeStruct((out_words,), jnp.float32),
        scratch_types=[
            pltpu.VMEM((g_per_w, GROUP), jnp.int32),
            pltpu.VMEM((SETB, GROUP, EMBED), jnp.float32),
            pltpu.VMEM((SETB, GROUP, EMBED), jnp.float32),
            pltpu.VMEM((SETW,), jnp.float32),
            pltpu.VMEM((SETW,), jnp.float32),
            pltpu.SemaphoreType.DMA,
            pltpu.SemaphoreType.DMA,
            pltpu.SemaphoreType.DMA,
            pltpu.SemaphoreType.DMA,
        ],
    )
    def emb(idx_hbm, table_hbm, out_hbm, idx_v, rows0, rows1, blk0, blk1,
            gsem0, gsem1, wsem0, wsem1):
        wid = lax.axis_index("s") * NC + lax.axis_index("c")
        gbase = wid * g_per_w
        pltpu.sync_copy(idx_hbm.at[pl.ds(gbase, g_per_w)], idx_v)

        lane = lax.iota(jnp.int32, 16)
        b_lo = (lane >> 3) * BLKW + (lane & 7) * GROUP

        def fire_set(s, rows_p, gsem_p):
            for b in range(SETB):
                pltpu.async_copy(
                    table_hbm.at[idx_v.at[s * SETB + b]],
                    rows_p.at[b],
                    gsem_p,
                )

        def drain_set(rows_p, gsem_p):
            for b in range(SETB):
                pltpu.make_async_copy(
                    table_hbm.at[pl.ds(0, GROUP)], rows_p.at[b], gsem_p
                ).wait()

        def do_set(it, p, rows_p, rows_q, blk_p, gsem_p, gsem_q, wsem_p):
            s = it * 2 + p
            g0 = gbase + s * SETB

            if p == 0:
                fire_set(s + 1, rows_q, gsem_q)
            else:
                @pl.when(it < n_pairs - 1)
                def _fire_next():
                    fire_set(s + 1, rows_q, gsem_q)

            drain_set(rows_p, gsem_p)

            @pl.when(it >= 1)
            def _wait_prev_writeback():
                pltpu.make_async_copy(
                    blk_p, out_hbm.at[pl.ds(0, SETW)], wsem_p
                ).wait()

            for b in range(SETB):
                def tbody(i2, carry):
                    off = b * (8 * GROUP) + i2
                    d_lo = rows_p[b, i2, pl.ds(0, 16)]
                    d_hi = rows_p[b, i2, pl.ds(16, 16)]
                    plsc.store_scatter(blk_p, [b_lo + off], d_lo)
                    plsc.store_scatter(blk_p, [b_lo + off + 2 * BLKW], d_hi)
                    return carry

                lax.fori_loop(0, GROUP, tbody, 0)

            j = g0 // ib_per_j
            ib0 = g0 % ib_per_j
            for eb in range(EMBED // 8):
                off = ((j * (EMBED // 8) + eb) * ib_per_j + ib0) * (8 * GROUP)
                pltpu.async_copy(
                    blk_p.at[pl.ds(eb * BLKW, BLKW)],
                    out_hbm.at[pl.ds(off, BLKW)],
                    wsem_p,
                )

        fire_set(0, rows0, gsem0)

        def body(it, carry):
            do_set(it, 0, rows0, rows1, blk0, gsem0, gsem1, wsem0)
            do_set(it, 1, rows1, rows0, blk1, gsem1, gsem0, wsem1)
            return carry

        lax.fori_loop(0, n_pairs, body, 0)

        pltpu.make_async_copy(blk0, out_hbm.at[pl.ds(0, SETW)], wsem0).wait()
        pltpu.make_async_copy(blk1, out_hbm.at[pl.ds(0, SETW)], wsem1).wait()

    out1d = emb(idx2d, table)
    return (
        out1d.reshape(NJ, EMBED // 8, ib_per_j, 8, GROUP)
        .transpose((2, 4, 0, 1, 3))
        .reshape(NI, NJ, EMBED)
    )
